# Initial kernel scaffold; baseline (speedup 1.0000x reference)
import jax
import jax.numpy as jnp
from jax.experimental import pallas as pl


def kernel(X, tables):
    raise NotImplementedError("write your pallas kernel here")


# speedup vs baseline: 1.3682x; 1.3682x over previous
import functools

import jax
import jax.numpy as jnp
from jax import lax
from jax.experimental import pallas as pl
from jax.experimental.pallas import tpu as pltpu, tpu_sc as plsc

B = 16384
F = 26
V = 100000

NC = 2
NS = 16
L = 16
NW = NC * NS
BPW = B // NW
CHUNK = 128
NCHUNK = BPW // CHUNK


def _body(x_hbm, table_hbm, out_hbm, xv, idxv, valsv, outv, sem):
    wid = lax.axis_index("s") * NC + lax.axis_index("c")
    base = wid * BPW

    pltpu.sync_copy(x_hbm.at[pl.ds(base * F, BPW * F)], xv)

    lane = lax.iota(jnp.int32, L)

    def build_f(f, _):
        fsplat = jnp.full((L,), f, jnp.int32)
        fbase = f * V

        def build_c(c, _):
            rows = c * L + lane
            vals = plsc.load_gather(xv, [rows * F + fsplat])
            j = c // (CHUNK // L)
            off = (c % (CHUNK // L)) * L
            idxv[f, j, pl.ds(off, L)] = vals + fbase
            return 0

        return lax.fori_loop(0, BPW // L, build_c, 0)

    lax.fori_loop(0, F, build_f, 0)

    def fire_f(f, _):
        def fire_c(j, _):
            pltpu.async_copy(
                table_hbm.at[idxv.at[f, j]],
                valsv.at[f, pl.ds(j * CHUNK, CHUNK)],
                sem,
            )
            return 0

        return lax.fori_loop(0, NCHUNK, fire_c, 0)

    lax.fori_loop(0, F, fire_f, 0)

    def drain_f(f, _):
        def drain_c(j, _):
            pltpu.make_async_copy(
                table_hbm.at[idxv.at[f, j]],
                valsv.at[f, pl.ds(j * CHUNK, CHUNK)],
                sem,
            ).wait()
            return 0

        return lax.fori_loop(0, NCHUNK, drain_c, 0)

    lax.fori_loop(0, F, drain_f, 0)

    def red_c(c, _):
        acc = jnp.zeros((L,), jnp.float32)
        for f in range(F):
            acc = acc + valsv[f, pl.ds(c * L, L)]
        outv[pl.ds(c * L, L)] = acc
        return 0

    lax.fori_loop(0, BPW // L, red_c, 0)

    pltpu.sync_copy(outv, out_hbm.at[pl.ds(base, BPW)])


@jax.jit
def _linear_logit(x, table_flat):
    mesh = plsc.VectorSubcoreMesh(core_axis_name="c", subcore_axis_name="s")
    return pl.kernel(
        _body,
        out_type=jax.ShapeDtypeStruct((B,), jnp.float32),
        mesh=mesh,
        compiler_params=pltpu.CompilerParams(needs_layout_passes=False),
        scratch_types=[
            pltpu.VMEM((BPW * F,), jnp.int32),
            pltpu.VMEM((F, NCHUNK, CHUNK), jnp.int32),
            pltpu.VMEM((F, BPW), jnp.float32),
            pltpu.VMEM((BPW,), jnp.float32),
            pltpu.SemaphoreType.DMA,
        ],
    )(x, table_flat)


def kernel(X, tables):
    x = X.astype(jnp.int32).reshape(B * F)
    table_flat = tables.reshape(F * V)
    return _linear_logit(x, table_flat)


# speedup vs baseline: 1.4944x; 1.0922x over previous
import jax
import jax.numpy as jnp
from jax import lax
from jax.experimental import pallas as pl
from jax.experimental.pallas import tpu as pltpu, tpu_sc as plsc

B = 16384
F = 26
V = 100000

NC = 2
NS = 16
L = 16
NW = NC * NS
BPW = B // NW
CHUNK = 128
NCHUNK = BPW // CHUNK


def _body(xt_hbm, table_hbm, out_hbm, idxv, valsv, outv, sem):
    wid = lax.axis_index("s") * NC + lax.axis_index("c")
    base = wid * BPW

    pltpu.sync_copy(xt_hbm.at[:, wid], idxv)

    def off_f(f, _):
        fbase = jnp.full((L,), f * V, jnp.int32)
        for j in range(NCHUNK):
            for o in range(CHUNK // L):
                sl = pl.ds(o * L, L)
                idxv[f, j, sl] = idxv[f, j, sl] + fbase
        return 0

    lax.fori_loop(0, F, off_f, 0)

    def fire_f(f, _):
        for j in range(NCHUNK):
            pltpu.async_copy(
                table_hbm.at[idxv.at[f, j]],
                valsv.at[f, pl.ds(j * CHUNK, CHUNK)],
                sem,
            )
        return 0

    lax.fori_loop(0, F, fire_f, 0)

    def drain_f(f, _):
        for j in range(NCHUNK):
            pltpu.make_async_copy(
                table_hbm.at[idxv.at[f, j]],
                valsv.at[f, pl.ds(j * CHUNK, CHUNK)],
                sem,
            ).wait()
        return 0

    lax.fori_loop(0, F, drain_f, 0)

    def red_c(c, _):
        acc = jnp.zeros((L,), jnp.float32)
        for f in range(F):
            acc = acc + valsv[f, pl.ds(c * L, L)]
        outv[pl.ds(c * L, L)] = acc
        return 0

    lax.fori_loop(0, BPW // L, red_c, 0)

    pltpu.sync_copy(outv, out_hbm.at[pl.ds(base, BPW)])


@jax.jit
def _linear_logit(xt, table_flat):
    mesh = plsc.VectorSubcoreMesh(core_axis_name="c", subcore_axis_name="s")
    return pl.kernel(
        _body,
        out_type=jax.ShapeDtypeStruct((B,), jnp.float32),
        mesh=mesh,
        compiler_params=pltpu.CompilerParams(needs_layout_passes=False),
        scratch_types=[
            pltpu.VMEM((F, NCHUNK, CHUNK), jnp.int32),
            pltpu.VMEM((F, BPW), jnp.float32),
            pltpu.VMEM((BPW,), jnp.float32),
            pltpu.SemaphoreType.DMA,
        ],
    )(xt, table_flat)


def kernel(X, tables):
    xt = X.astype(jnp.int32).T.reshape(F, NW, NCHUNK, CHUNK)
    table_flat = tables.reshape(F * V)
    return _linear_logit(xt, table_flat)
